# baseline (speedup 1.0000x reference)
import functools

import jax
import jax.numpy as jnp
from jax.experimental import pallas as pl
from jax.experimental.pallas import tpu as pltpu

_C = 16
_UNK = 1


def _fused_kernel(sid_ref, qid_ref, table_ref, wih_ref, whh_ref, b_ref,
                  mask_ref, sims_ref, top_ref, hall_ref, xbuf_ref,
                  *, E, S, GB, V):
    C = _C
    n = GB * (C + 1)
    nsrc = GB * C
    g = pl.program_id(0)

    def gather_step(tg, buf):
        for r in range(n):
            if r < nsrc:
                addr = g * (nsrc * S) + (r * S) + tg
                raw = sid_ref[addr >> 7, addr & 127]
            else:
                addr = g * (GB * S) + ((r - nsrc) * S) + tg
                raw = qid_ref[addr >> 7, addr & 127]
            tok = jnp.where(raw >= V, _UNK, raw)
            tok2 = pl.multiple_of(tok * 2, 2)
            xbuf_ref[buf, pl.ds(2 * r, 2), :] = table_ref[pl.ds(tok2, 2), :]

    gather_step(0, 0)
    gather_step(1, 1)

    wih = wih_ref[...]
    whh = whh_ref[...]
    bias = b_ref[...]

    def sigm(v):
        return 0.5 * jnp.tanh(0.5 * v) + 0.5

    def lstm_pair(k, carry):
        h, c = carry
        for j in range(2):
            t = 2 * k + j
            x_t = xbuf_ref[t & 3].reshape(n, E)
            gather_step(t + 2, (t + 2) & 3)
            gates = (jnp.dot(x_t, wih, preferred_element_type=jnp.float32)
                     + jnp.dot(h, whh, preferred_element_type=jnp.float32)
                     + bias)
            sig_if = sigm(gates[:, :2 * E])
            g_g = jnp.tanh(gates[:, 2 * E:3 * E])
            o_g = sigm(gates[:, 3 * E:])
            i_g = sig_if[:, :E]
            f_g = sig_if[:, E:]
            c = f_g * c + i_g * g_g
            h = o_g * jnp.tanh(c)
            off = pl.multiple_of(t * E, E)
            hall_ref[:, pl.ds(off, E)] = h.astype(hall_ref.dtype)
        return h, c

    h0 = jnp.zeros((n, E), jnp.float32)
    c0 = jnp.zeros((n, E), jnp.float32)
    jax.lax.fori_loop(0, S // 2, lstm_pair, (h0, c0))

    qmask = mask_ref[...].astype(jnp.float32)
    qm = hall_ref[nsrc:n, :].astype(jnp.float32) * qmask

    s = jnp.zeros((GB, C), jnp.float32)
    KCH = min(2048, S * E)
    for j in range(0, S * E, KCH):
        src_j = hall_ref[:nsrc, j:j + KCH].astype(jnp.float32)
        src_j = src_j.reshape(GB, C, KCH)
        s = s + jnp.sum(src_j * qm[:, None, j:j + KCH], axis=-1)

    m = jnp.max(s, axis=-1, keepdims=True)
    e = jnp.exp(s - m)
    sims_ref[...] = e / jnp.sum(e, axis=-1, keepdims=True)
    idx = jax.lax.broadcasted_iota(jnp.int32, s.shape, 1)
    top_ref[...] = jnp.min(jnp.where(s == m, idx, jnp.int32(C)),
                           axis=-1, keepdims=True)


def kernel(sources, queries, embedding, w_ih, w_hh, b):
    C = _C
    B, S = queries.shape
    V, E = embedding.shape
    GB = 16 if B % 16 == 0 else (8 if B % 8 == 0 else B)

    pad = jnp.zeros((1, 128), jnp.int32)
    sid = jnp.concatenate([sources.reshape(-1, 128), pad], axis=0)
    qid = jnp.concatenate([queries.reshape(-1, 128), pad], axis=0)
    table2 = embedding.reshape(2 * V, E // 2)

    q_len = jnp.sum((queries > 0).astype(jnp.int32), axis=1)
    mask = jnp.arange(S)[None, :] < q_len[:, None]
    mask_flat = (jnp.broadcast_to(mask[:, :, None], (B, S, E))
                 .reshape(B, S * E).astype(jnp.bfloat16))

    grid = (B // GB,)
    n_rows = GB * (C + 1)
    body = functools.partial(_fused_kernel, E=E, S=S, GB=GB, V=V)
    sims, top = pl.pallas_call(
        body,
        out_shape=(jax.ShapeDtypeStruct((B, C), jnp.float32),
                   jax.ShapeDtypeStruct((B, 1), jnp.int32)),
        grid_spec=pltpu.PrefetchScalarGridSpec(
            num_scalar_prefetch=2,
            grid=grid,
            in_specs=[
                pl.BlockSpec((2 * V, E // 2), lambda g, s_, q_: (0, 0)),
                pl.BlockSpec((E, 4 * E), lambda g, s_, q_: (0, 0)),
                pl.BlockSpec((E, 4 * E), lambda g, s_, q_: (0, 0)),
                pl.BlockSpec((1, 4 * E), lambda g, s_, q_: (0, 0)),
                pl.BlockSpec((GB, S * E), lambda g, s_, q_: (g, 0)),
            ],
            out_specs=(pl.BlockSpec((GB, C), lambda g, s_, q_: (g, 0)),
                       pl.BlockSpec((GB, 1), lambda g, s_, q_: (g, 0))),
            scratch_shapes=[
                pltpu.VMEM((n_rows, S * E), jnp.bfloat16),
                pltpu.VMEM((4, 2 * n_rows, E // 2), jnp.float32),
            ],
        ),
        compiler_params=pltpu.CompilerParams(
            dimension_semantics=("parallel",),
            vmem_limit_bytes=100 * 1024 * 1024,
        ),
    )(sid, qid, table2, w_ih, w_hh, b, mask_flat)

    offsets = jnp.arange(B, dtype=jnp.int32) * C
    selected = jnp.take(sources, offsets + top[:, 0], axis=0)
    return selected, sims


# speedup vs baseline: 1.0446x; 1.0446x over previous
import functools

import jax
import jax.numpy as jnp
from jax.experimental import pallas as pl
from jax.experimental.pallas import tpu as pltpu

_C = 16
_UNK = 1


def _fused_kernel(sid_ref, qid_ref, table_ref, wih_ref, whh_ref, b_ref,
                  mask_ref, sims_ref, top_ref, hall_ref, xbuf_ref,
                  *, E, S, GB, V):
    C = _C
    n = GB * (C + 1)
    nsrc = GB * C
    g = pl.program_id(0)

    def gather_step(tg, buf):
        for r in range(n):
            if r < nsrc:
                addr = g * (nsrc * S) + (r * S) + tg
                raw = sid_ref[addr >> 7, addr & 127]
            else:
                addr = g * (GB * S) + ((r - nsrc) * S) + tg
                raw = qid_ref[addr >> 7, addr & 127]
            tok = jnp.where(raw >= V, _UNK, raw)
            tok2 = pl.multiple_of(tok * 2, 2)
            xbuf_ref[buf, pl.ds(2 * r, 2), :] = table_ref[pl.ds(tok2, 2), :]

    gather_step(0, 0)
    gather_step(1, 1)

    wih = wih_ref[...]
    whh = whh_ref[...]
    bias = b_ref[...]

    def lstm_step(t, carry):
        h, c = carry
        x_t = xbuf_ref[t & 3].reshape(n, E)
        gather_step(t + 2, (t + 2) & 3)
        gates = (jnp.dot(x_t, wih, preferred_element_type=jnp.float32)
                 + jnp.dot(h, whh, preferred_element_type=jnp.float32)
                 + bias)
        sig_if = jax.nn.sigmoid(gates[:, :2 * E])
        g_g = jnp.tanh(gates[:, 2 * E:3 * E])
        o_g = jax.nn.sigmoid(gates[:, 3 * E:])
        i_g = sig_if[:, :E]
        f_g = sig_if[:, E:]
        c = f_g * c + i_g * g_g
        h = o_g * jnp.tanh(c)
        off = pl.multiple_of(t * E, E)
        hall_ref[:, pl.ds(off, E)] = h.astype(hall_ref.dtype)
        return h, c

    h0 = jnp.zeros((n, E), jnp.float32)
    c0 = jnp.zeros((n, E), jnp.float32)
    jax.lax.fori_loop(0, S, lstm_step, (h0, c0))

    qmask = mask_ref[...].astype(jnp.float32)
    qm = hall_ref[nsrc:n, :].astype(jnp.float32) * qmask

    s = jnp.zeros((GB, C), jnp.float32)
    KCH = min(2048, S * E)
    for j in range(0, S * E, KCH):
        src_j = hall_ref[:nsrc, j:j + KCH].astype(jnp.float32)
        src_j = src_j.reshape(GB, C, KCH)
        s = s + jnp.sum(src_j * qm[:, None, j:j + KCH], axis=-1)

    m = jnp.max(s, axis=-1, keepdims=True)
    e = jnp.exp(s - m)
    sims_ref[...] = e / jnp.sum(e, axis=-1, keepdims=True)
    idx = jax.lax.broadcasted_iota(jnp.int32, s.shape, 1)
    top_ref[...] = jnp.min(jnp.where(s == m, idx, jnp.int32(C)),
                           axis=-1, keepdims=True)


def kernel(sources, queries, embedding, w_ih, w_hh, b):
    C = _C
    B, S = queries.shape
    V, E = embedding.shape
    GB = 16 if B % 16 == 0 else (8 if B % 8 == 0 else B)

    pad = jnp.zeros((1, 128), jnp.int32)
    sid = jnp.concatenate([sources.reshape(-1, 128), pad], axis=0)
    qid = jnp.concatenate([queries.reshape(-1, 128), pad], axis=0)
    table2 = embedding.reshape(2 * V, E // 2)

    q_len = jnp.sum((queries > 0).astype(jnp.int32), axis=1)
    mask = jnp.arange(S)[None, :] < q_len[:, None]
    mask_flat = (jnp.broadcast_to(mask[:, :, None], (B, S, E))
                 .reshape(B, S * E).astype(jnp.bfloat16))

    grid = (B // GB,)
    n_rows = GB * (C + 1)
    body = functools.partial(_fused_kernel, E=E, S=S, GB=GB, V=V)
    sims, top = pl.pallas_call(
        body,
        out_shape=(jax.ShapeDtypeStruct((B, C), jnp.float32),
                   jax.ShapeDtypeStruct((B, 1), jnp.int32)),
        grid_spec=pltpu.PrefetchScalarGridSpec(
            num_scalar_prefetch=2,
            grid=grid,
            in_specs=[
                pl.BlockSpec((2 * V, E // 2), lambda g, s_, q_: (0, 0)),
                pl.BlockSpec((E, 4 * E), lambda g, s_, q_: (0, 0)),
                pl.BlockSpec((E, 4 * E), lambda g, s_, q_: (0, 0)),
                pl.BlockSpec((1, 4 * E), lambda g, s_, q_: (0, 0)),
                pl.BlockSpec((GB, S * E), lambda g, s_, q_: (g, 0)),
            ],
            out_specs=(pl.BlockSpec((GB, C), lambda g, s_, q_: (g, 0)),
                       pl.BlockSpec((GB, 1), lambda g, s_, q_: (g, 0))),
            scratch_shapes=[
                pltpu.VMEM((n_rows, S * E), jnp.bfloat16),
                pltpu.VMEM((4, 2 * n_rows, E // 2), jnp.float32),
            ],
        ),
        compiler_params=pltpu.CompilerParams(
            dimension_semantics=("parallel",),
            vmem_limit_bytes=100 * 1024 * 1024,
        ),
    )(sid, qid, table2, w_ih, w_hh, b, mask_flat)

    offsets = jnp.arange(B, dtype=jnp.int32) * C
    selected = jnp.take(sources, offsets + top[:, 0], axis=0)
    return selected, sims


# speedup vs baseline: 1.0841x; 1.0378x over previous
import functools

import jax
import jax.numpy as jnp
from jax.experimental import pallas as pl
from jax.experimental.pallas import tpu as pltpu

_C = 16
_UNK = 1


def _fused_kernel(sid_ref, qid_ref, table_ref, wih_ref, whh_ref, b_ref,
                  mask_ref, sims_ref, top_ref, hall_ref, xbuf_ref,
                  *, E, S, GB, V):
    C = _C
    n = GB * (C + 1)
    nsrc = GB * C
    g = pl.program_id(0)

    def gather_step(tg, buf):
        for r in range(n):
            if r < nsrc:
                addr = g * (nsrc * S) + (r * S) + tg
                raw = sid_ref[addr >> 7, addr & 127]
            else:
                addr = g * (GB * S) + ((r - nsrc) * S) + tg
                raw = qid_ref[addr >> 7, addr & 127]
            tok = jnp.where(raw >= V, _UNK, raw)
            tok2 = pl.multiple_of(tok * 2, 2)
            xbuf_ref[buf, pl.ds(2 * r, 2), :] = table_ref[pl.ds(tok2, 2), :]

    gather_step(0, 0)
    gather_step(1, 1)

    wih = wih_ref[...]
    whh = whh_ref[...]
    bias = b_ref[...]

    def lstm_step(t, carry):
        h, c = carry
        x_t = xbuf_ref[t & 3].reshape(n, E)
        gather_step(t + 2, (t + 2) & 3)
        gates = (jnp.dot(x_t, wih, preferred_element_type=jnp.float32)
                 + jnp.dot(h, whh, preferred_element_type=jnp.float32)
                 + bias)
        sig_if = jax.nn.sigmoid(gates[:, :2 * E])
        g_g = jnp.tanh(gates[:, 2 * E:3 * E])
        o_g = jax.nn.sigmoid(gates[:, 3 * E:])
        i_g = sig_if[:, :E]
        f_g = sig_if[:, E:]
        c = f_g * c + i_g * g_g
        h = o_g * jnp.tanh(c)
        off = pl.multiple_of(t * E, E)
        hall_ref[:, pl.ds(off, E)] = h.astype(hall_ref.dtype)
        return h, c

    h0 = jnp.zeros((n, E), jnp.float32)
    c0 = jnp.zeros((n, E), jnp.float32)
    jax.lax.fori_loop(0, S, lstm_step, (h0, c0))

    qmask = mask_ref[...].astype(jnp.float32)
    qm = hall_ref[nsrc:n, :].astype(jnp.float32) * qmask

    s = jnp.zeros((GB, C), jnp.float32)
    KCH = min(2048, S * E)
    for j in range(0, S * E, KCH):
        src_j = hall_ref[:nsrc, j:j + KCH].astype(jnp.float32)
        src_j = src_j.reshape(GB, C, KCH)
        s = s + jnp.sum(src_j * qm[:, None, j:j + KCH], axis=-1)

    m = jnp.max(s, axis=-1, keepdims=True)
    e = jnp.exp(s - m)
    sims_ref[...] = e / jnp.sum(e, axis=-1, keepdims=True)
    idx = jax.lax.broadcasted_iota(jnp.int32, s.shape, 1)
    top_ref[...] = jnp.min(jnp.where(s == m, idx, jnp.int32(C)),
                           axis=-1, keepdims=True)


def kernel(sources, queries, embedding, w_ih, w_hh, b):
    C = _C
    B, S = queries.shape
    V, E = embedding.shape
    GB = 32 if B % 32 == 0 else (8 if B % 8 == 0 else B)

    pad = jnp.zeros((1, 128), jnp.int32)
    sid = jnp.concatenate([sources.reshape(-1, 128), pad], axis=0)
    qid = jnp.concatenate([queries.reshape(-1, 128), pad], axis=0)
    table2 = embedding.reshape(2 * V, E // 2)

    q_len = jnp.sum((queries > 0).astype(jnp.int32), axis=1)
    mask = jnp.arange(S)[None, :] < q_len[:, None]
    mask_flat = (jnp.broadcast_to(mask[:, :, None], (B, S, E))
                 .reshape(B, S * E).astype(jnp.bfloat16))

    grid = (B // GB,)
    n_rows = GB * (C + 1)
    body = functools.partial(_fused_kernel, E=E, S=S, GB=GB, V=V)
    sims, top = pl.pallas_call(
        body,
        out_shape=(jax.ShapeDtypeStruct((B, C), jnp.float32),
                   jax.ShapeDtypeStruct((B, 1), jnp.int32)),
        grid_spec=pltpu.PrefetchScalarGridSpec(
            num_scalar_prefetch=2,
            grid=grid,
            in_specs=[
                pl.BlockSpec((2 * V, E // 2), lambda g, s_, q_: (0, 0)),
                pl.BlockSpec((E, 4 * E), lambda g, s_, q_: (0, 0)),
                pl.BlockSpec((E, 4 * E), lambda g, s_, q_: (0, 0)),
                pl.BlockSpec((1, 4 * E), lambda g, s_, q_: (0, 0)),
                pl.BlockSpec((GB, S * E), lambda g, s_, q_: (g, 0)),
            ],
            out_specs=(pl.BlockSpec((GB, C), lambda g, s_, q_: (g, 0)),
                       pl.BlockSpec((GB, 1), lambda g, s_, q_: (g, 0))),
            scratch_shapes=[
                pltpu.VMEM((n_rows, S * E), jnp.bfloat16),
                pltpu.VMEM((4, 2 * n_rows, E // 2), jnp.float32),
            ],
        ),
        compiler_params=pltpu.CompilerParams(
            dimension_semantics=("parallel",),
            vmem_limit_bytes=100 * 1024 * 1024,
        ),
    )(sid, qid, table2, w_ih, w_hh, b, mask_flat)

    offsets = jnp.arange(B, dtype=jnp.int32) * C
    selected = jnp.take(sources, offsets + top[:, 0], axis=0)
    return selected, sims
